# Initial kernel scaffold; baseline (speedup 1.0000x reference)
import jax
import jax.numpy as jnp
from jax.experimental import pallas as pl


def kernel(times, num_obs, X, M, delta_t, cov, val_times, params):
    raise NotImplementedError("write your pallas kernel here")


# speedup vs baseline: 5.3595x; 5.3595x over previous
import math

import jax
import jax.numpy as jnp
from jax.experimental import pallas as pl

B, L, LV = 16, 256, 4
D = 64
HID = 256
PHID = 128
PREP = 16
MIX = 0.1
LOG_LIK_C = math.log(math.sqrt(2.0 * math.pi))
OBS_STD = 0.01
LOG_S2 = math.log(OBS_STD)
INV_2S2 = 1.0 / (2.0 * OBS_STD * OBS_STD)
LVP = 8


def _fwd_kernel(cov_ref, covW1_ref, covb1_ref, covW2_ref, covb2_ref,
                Wrz_ref, Whh_ref, Wpg_ref, bpg_ref, pW2_ref, pb2_ref,
                A_ref, bprep_ref, R_ref, Wih_ref, bih_ref,
                X_ref, M_ref, dtm_ref, dtv_ref,
                h_ref, l1_ref, l2_ref):
    a = jnp.maximum(cov_ref[...] @ covW1_ref[...] + covb1_ref[...], 0.0)
    h0 = jnp.tanh(a @ covW2_ref[...] + covb2_ref[...])

    Wrz = Wrz_ref[...]
    Whh = Whh_ref[...]
    Wpg = Wpg_ref[...]
    bpg = bpg_ref[...]
    pW2 = pW2_ref[...]
    pb2 = pb2_ref[...]
    A = A_ref[...]
    bprep = bprep_ref[...]
    R = R_ref[...]
    Wih = Wih_ref[...]
    bih = bih_ref[...]
    two_c = 2.0 * LOG_LIK_C

    def step(t, carry):
        h, acc1, acc2 = carry
        Xt = X_ref[t]
        Mt = M_ref[t]
        diff = dtm_ref[t]
        rz = jax.nn.sigmoid(h @ Wrz)
        r = rz[:, :HID]
        z = rz[:, HID:]
        u = jnp.tanh((r * h) @ Whh)
        h = h + (1.0 - z) * (u - h) * diff
        pg = h @ Wpg + bpg
        a1 = jnp.maximum(pg[:, :PHID], 0.0)
        gh = pg[:, PHID:]
        p = a1 @ pW2 + pb2
        mean = p[:, :D]
        logvar = p[:, D:]
        error = (Xt - mean) / jnp.exp(0.5 * logvar)
        acc1 = acc1 + (error * error + logvar + two_c) * Mt
        G = jnp.concatenate([Xt, mean, logvar, error], axis=1)
        gi = jnp.maximum(G @ A + bprep, 0.0) * (Mt @ R)
        gg = gi @ Wih + bih
        rg = jax.nn.sigmoid(gg[:, :HID] + gh[:, :HID])
        zg = jax.nn.sigmoid(gg[:, HID:2 * HID] + gh[:, HID:2 * HID])
        n = jnp.tanh(gg[:, 2 * HID:] + rg * gh[:, 2 * HID:])
        h = (1.0 - zg) * n + zg * h
        a2 = jnp.maximum(h @ Wpg[:, :PHID] + bpg[:, :PHID], 0.0)
        p2 = a2 @ pW2 + pb2
        m2 = p2[:, :D]
        v2 = p2[:, D:]
        kl = (LOG_S2 - 0.5) - 0.5 * v2 + (jnp.exp(v2) + jnp.square(m2 - Xt)) * INV_2S2
        acc2 = acc2 + kl * Mt
        return (h, acc1, acc2)

    zero = jnp.zeros((B, D), jnp.float32)
    h, acc1, acc2 = jax.lax.fori_loop(0, L, step, (h0, zero, zero))

    def vstep(j, h):
        diff = dtv_ref[j]
        rz = jax.nn.sigmoid(h @ Wrz)
        r = rz[:, :HID]
        z = rz[:, HID:]
        u = jnp.tanh((r * h) @ Whh)
        return h + (1.0 - z) * (u - h) * diff

    h = jax.lax.fori_loop(0, LVP, vstep, h)

    h_ref[...] = h
    l1_ref[...] = jnp.reshape(0.5 * jnp.sum(acc1), (1, 1))
    l2_ref[...] = jnp.reshape(jnp.sum(acc2), (1, 1))


def kernel(times, num_obs, X, M, delta_t, cov, val_times, params):
    p = params
    f32 = jnp.float32
    dtm = jnp.concatenate([times[:, :1], times[:, 1:] - times[:, :-1]], axis=1)
    dtm = dtm.T[:, :, None]
    vt = jnp.concatenate([times[:, -1:], val_times], axis=1)
    dtv = (vt[:, 1:] - vt[:, :-1]).T[:, :, None]
    dtv = jnp.concatenate([dtv, jnp.zeros((LVP - LV, B, 1), f32)], axis=0)

    X3 = X.reshape(B, L, D).transpose(1, 0, 2)
    M3 = M.reshape(B, L, D).transpose(1, 0, 2)

    Wrz = jnp.concatenate([p['ode_Whr'], p['ode_Whz']], axis=1)
    Wpg = jnp.concatenate([p['p_W1'], p['gru_Whh']], axis=1)
    bpg = jnp.concatenate([p['p_b1'], p['gru_bhh']])[None, :]

    eye = jnp.eye(D, dtype=f32)
    wp = p['w_prep']
    blocks = [(eye[:, :, None] * wp[None, :, k, :]).reshape(D, D * PREP)
              for k in range(4)]
    A = jnp.concatenate(blocks, axis=0)
    R = jnp.repeat(eye, PREP, axis=1)
    bprep = p['bias_prep'].reshape(1, D * PREP)

    out_shapes = [
        jax.ShapeDtypeStruct((B, HID), f32),
        jax.ShapeDtypeStruct((1, 1), f32),
        jax.ShapeDtypeStruct((1, 1), f32),
    ]
    h, l1, l2 = pl.pallas_call(_fwd_kernel, out_shape=out_shapes)(
        cov, p['cov_W1'], p['cov_b1'][None, :], p['cov_W2'], p['cov_b2'][None, :],
        Wrz, p['ode_Whh'], Wpg, bpg, p['p_W2'], p['p_b2'][None, :],
        A, bprep, R, p['gru_Wih'], p['gru_bih'][None, :],
        X3, M3, dtm, dtv)
    l1 = l1[0, 0]
    l2 = l2[0, 0]
    loss = l1 + MIX * l2
    nll = l1 / (L * B * D)
    return h, loss, nll, l1, l2


# speedup vs baseline: 6.5503x; 1.2222x over previous
import math

import jax
import jax.numpy as jnp
from jax.experimental import pallas as pl

B, L, LV = 16, 256, 4
D = 64
HID = 256
PHID = 128
PREP = 16
MIX = 0.1
LOG_LIK_C = math.log(math.sqrt(2.0 * math.pi))
OBS_STD = 0.01
LOG_S2 = math.log(OBS_STD)
INV_2S2 = 1.0 / (2.0 * OBS_STD * OBS_STD)
LVP = 8


def _fwd_kernel(cov_ref, covW1_ref, covb1_ref, covW2_ref, covb2_ref,
                Wpc_ref, Whh_ref, Wpg_ref, bpg_ref, pW2_ref, pb1_ref, pb2_ref,
                A_ref, Wih_ref, bih_ref,
                X_ref, M_ref, dtm_ref, dtv_ref,
                h_ref, l1_ref, l2_ref):
    a = jnp.maximum(cov_ref[...] @ covW1_ref[...] + covb1_ref[...], 0.0)
    h0 = jnp.tanh(a @ covW2_ref[...] + covb2_ref[...])

    Wpc = Wpc_ref[...]
    Whh = Whh_ref[...]
    Wpg = Wpg_ref[...]
    bpg = bpg_ref[...]
    pW2 = pW2_ref[...]
    pb1 = pb1_ref[...]
    pb2 = pb2_ref[...]
    A = A_ref[...]
    Wih = Wih_ref[...]
    bih = bih_ref[...]
    two_c = 2.0 * LOG_LIK_C

    def kl_term(pc, Xp, Mp):
        a2 = jnp.maximum(pc[:, :PHID] + pb1, 0.0)
        p2 = a2 @ pW2 + pb2
        m2 = p2[:, :D]
        v2 = p2[:, D:]
        kl = (LOG_S2 - 0.5) - 0.5 * v2 \
            + (jnp.exp(v2) + jnp.square(m2 - Xp)) * INV_2S2
        return kl * Mp

    def step(t, carry):
        h, pc, acc1, acc2 = carry
        tp = jnp.maximum(t - 1, 0)
        valid = jnp.where(t > 0, 1.0, 0.0)
        acc2 = acc2 + kl_term(pc, X_ref[tp], M_ref[tp]) * valid
        Xt = X_ref[t]
        Mt = M_ref[t]
        rz = jax.nn.sigmoid(pc[:, PHID:])
        r = rz[:, :HID]
        z = rz[:, HID:]
        u = jnp.tanh((r * h) @ Whh)
        h = h + (1.0 - z) * (u - h) * dtm_ref[t]
        pg = h @ Wpg + bpg
        a1 = jnp.maximum(pg[:, :PHID], 0.0)
        gh = pg[:, PHID:]
        p = a1 @ pW2 + pb2
        mean = p[:, :D]
        logvar = p[:, D:]
        error = (Xt - mean) / jnp.exp(0.5 * logvar)
        acc1 = acc1 + (error * error + logvar + two_c) * Mt
        G = jnp.concatenate(
            [Xt * Mt, mean * Mt, logvar * Mt, error * Mt, Mt], axis=1)
        gi = jnp.maximum(G @ A, 0.0)
        gg = gi @ Wih + bih
        rg = jax.nn.sigmoid(gg[:, :HID] + gh[:, :HID])
        zg = jax.nn.sigmoid(gg[:, HID:2 * HID] + gh[:, HID:2 * HID])
        n = jnp.tanh(gg[:, 2 * HID:] + rg * gh[:, 2 * HID:])
        h = (1.0 - zg) * n + zg * h
        pc = h @ Wpc
        return (h, pc, acc1, acc2)

    zero = jnp.zeros((B, D), jnp.float32)
    pc0 = h0 @ Wpc
    h, pc, acc1, acc2 = jax.lax.fori_loop(0, L, step, (h0, pc0, zero, zero))
    acc2 = acc2 + kl_term(pc, X_ref[L - 1], M_ref[L - 1])

    def vstep(j, carry):
        h, pc = carry
        rz = jax.nn.sigmoid(pc[:, PHID:])
        r = rz[:, :HID]
        z = rz[:, HID:]
        u = jnp.tanh((r * h) @ Whh)
        h = h + (1.0 - z) * (u - h) * dtv_ref[j]
        return (h, h @ Wpc)

    h, _ = jax.lax.fori_loop(0, LVP, vstep, (h, pc))

    h_ref[...] = h
    l1_ref[...] = jnp.reshape(0.5 * jnp.sum(acc1), (1, 1))
    l2_ref[...] = jnp.reshape(jnp.sum(acc2), (1, 1))


def kernel(times, num_obs, X, M, delta_t, cov, val_times, params):
    p = params
    f32 = jnp.float32
    dtm = jnp.concatenate([times[:, :1], times[:, 1:] - times[:, :-1]], axis=1)
    dtm = dtm.T[:, :, None]
    vt = jnp.concatenate([times[:, -1:], val_times], axis=1)
    dtv = (vt[:, 1:] - vt[:, :-1]).T[:, :, None]
    dtv = jnp.concatenate([dtv, jnp.zeros((LVP - LV, B, 1), f32)], axis=0)

    X3 = X.reshape(B, L, D).transpose(1, 0, 2)
    M3 = M.reshape(B, L, D).transpose(1, 0, 2)

    Wpc = jnp.concatenate([p['p_W1'], p['ode_Whr'], p['ode_Whz']], axis=1)
    Wpg = jnp.concatenate([p['p_W1'], p['gru_Whh']], axis=1)
    bpg = jnp.concatenate([p['p_b1'], p['gru_bhh']])[None, :]

    eye = jnp.eye(D, dtype=f32)
    wp = p['w_prep']
    blocks = [(eye[:, :, None] * wp[None, :, k, :]).reshape(D, D * PREP)
              for k in range(4)]
    blocks.append((eye[:, :, None] * p['bias_prep'][None, :, :]).reshape(D, D * PREP))
    A = jnp.concatenate(blocks, axis=0)

    out_shapes = [
        jax.ShapeDtypeStruct((B, HID), f32),
        jax.ShapeDtypeStruct((1, 1), f32),
        jax.ShapeDtypeStruct((1, 1), f32),
    ]
    h, l1, l2 = pl.pallas_call(_fwd_kernel, out_shape=out_shapes)(
        cov, p['cov_W1'], p['cov_b1'][None, :], p['cov_W2'], p['cov_b2'][None, :],
        Wpc, p['ode_Whh'], Wpg, bpg, p['p_W2'], p['p_b1'][None, :], p['p_b2'][None, :],
        A, p['gru_Wih'], p['gru_bih'][None, :],
        X3, M3, dtm, dtv)
    l1 = l1[0, 0]
    l2 = l2[0, 0]
    loss = l1 + MIX * l2
    nll = l1 / (L * B * D)
    return h, loss, nll, l1, l2


# speedup vs baseline: 6.7443x; 1.0296x over previous
import math

import jax
import jax.numpy as jnp
from jax.experimental import pallas as pl
from jax.experimental.pallas import tpu as pltpu

B, L, LV = 16, 256, 4
D = 64
HID = 256
PHID = 128
PREP = 16
MIX = 0.1
LOG_LIK_C = math.log(math.sqrt(2.0 * math.pi))
OBS_STD = 0.01
LOG_S2 = math.log(OBS_STD)
INV_2S2 = 1.0 / (2.0 * OBS_STD * OBS_STD)
LVP = 8


def _fwd_kernel(cov_ref, covW1_ref, covb1_ref, covW2_ref, covb2_ref,
                Wrz_ref, Whh_ref, Wpg_ref, bpg_ref,
                pW1_ref, pb1_ref, pW2_ref, pb2_ref,
                Axm_ref, Ap_ref, Wih_ref, bih_ref,
                X_ref, M_ref, dtm_ref, dtv_ref,
                h_ref, l1_ref, l2_ref, Hall_ref):
    a = jnp.maximum(cov_ref[...] @ covW1_ref[...] + covb1_ref[...], 0.0)
    h0 = jnp.tanh(a @ covW2_ref[...] + covb2_ref[...])

    Wrz = Wrz_ref[...]
    Whh = Whh_ref[...]
    Wpg = Wpg_ref[...]
    bpg = bpg_ref[...]
    pW2 = pW2_ref[...]
    pb2 = pb2_ref[...]
    Axm = Axm_ref[...]
    Ap = Ap_ref[...]
    Wih = Wih_ref[...]
    bih = bih_ref[...]
    two_c = 2.0 * LOG_LIK_C

    def step(t, carry):
        h, pc, acc1 = carry
        Xt = X_ref[t]
        Mt = M_ref[t]
        base = jnp.concatenate([Xt * Mt, Mt], axis=1) @ Axm
        rz = jax.nn.sigmoid(pc)
        r = rz[:, :HID]
        z = rz[:, HID:]
        u = jnp.tanh((r * h) @ Whh)
        h = h + (1.0 - z) * (u - h) * dtm_ref[t]
        pg = h @ Wpg + bpg
        a1 = jnp.maximum(pg[:, :PHID], 0.0)
        gh = pg[:, PHID:]
        p = a1 @ pW2 + pb2
        mean = p[:, :D]
        logvar = p[:, D:]
        error = (Xt - mean) * jnp.exp(-0.5 * logvar)
        acc1 = acc1 + (error * error + logvar + two_c) * Mt
        Gp = jnp.concatenate([mean * Mt, logvar * Mt, error * Mt], axis=1)
        gi = jnp.maximum(base + Gp @ Ap, 0.0)
        gg = gi @ Wih + bih
        rg = jax.nn.sigmoid(gg[:, :HID] + gh[:, :HID])
        zg = jax.nn.sigmoid(gg[:, HID:2 * HID] + gh[:, HID:2 * HID])
        n = jnp.tanh(gg[:, 2 * HID:] + rg * gh[:, 2 * HID:])
        h = (1.0 - zg) * n + zg * h
        Hall_ref[t] = h
        pc = h @ Wrz
        return (h, pc, acc1)

    zero = jnp.zeros((B, D), jnp.float32)
    pc0 = h0 @ Wrz
    h, pc, acc1 = jax.lax.fori_loop(0, L, step, (h0, pc0, zero))

    def vstep(j, carry):
        h, pc = carry
        rz = jax.nn.sigmoid(pc)
        r = rz[:, :HID]
        z = rz[:, HID:]
        u = jnp.tanh((r * h) @ Whh)
        h = h + (1.0 - z) * (u - h) * dtv_ref[j]
        return (h, h @ Wrz)

    hf, _ = jax.lax.fori_loop(0, LVP, vstep, (h, pc))
    h_ref[...] = hf

    H = Hall_ref[...].reshape(L * B, HID)
    A2 = jnp.maximum(H @ pW1_ref[...] + pb1_ref[...], 0.0)
    P2 = A2 @ pW2 + pb2
    m2 = P2[:, :D]
    v2 = P2[:, D:]
    Xf = X_ref[...].reshape(L * B, D)
    Mf = M_ref[...].reshape(L * B, D)
    kl = (LOG_S2 - 0.5) - 0.5 * v2 + (jnp.exp(v2) + jnp.square(m2 - Xf)) * INV_2S2
    l1_ref[...] = jnp.reshape(0.5 * jnp.sum(acc1), (1, 1))
    l2_ref[...] = jnp.reshape(jnp.sum(kl * Mf), (1, 1))


def kernel(times, num_obs, X, M, delta_t, cov, val_times, params):
    p = params
    f32 = jnp.float32
    dtm = jnp.concatenate([times[:, :1], times[:, 1:] - times[:, :-1]], axis=1)
    dtm = dtm.T[:, :, None]
    vt = jnp.concatenate([times[:, -1:], val_times], axis=1)
    dtv = (vt[:, 1:] - vt[:, :-1]).T[:, :, None]
    dtv = jnp.concatenate([dtv, jnp.zeros((LVP - LV, B, 1), f32)], axis=0)

    X3 = X.reshape(B, L, D).transpose(1, 0, 2)
    M3 = M.reshape(B, L, D).transpose(1, 0, 2)

    Wrz = jnp.concatenate([p['ode_Whr'], p['ode_Whz']], axis=1)
    Wpg = jnp.concatenate([p['p_W1'], p['gru_Whh']], axis=1)
    bpg = jnp.concatenate([p['p_b1'], p['gru_bhh']])[None, :]

    eye = jnp.eye(D, dtype=f32)
    wp = p['w_prep']
    blk = [(eye[:, :, None] * wp[None, :, k, :]).reshape(D, D * PREP)
           for k in range(4)]
    bblk = (eye[:, :, None] * p['bias_prep'][None, :, :]).reshape(D, D * PREP)
    Axm = jnp.concatenate([blk[0], bblk], axis=0)
    Ap = jnp.concatenate([blk[1], blk[2], blk[3]], axis=0)

    out_shapes = [
        jax.ShapeDtypeStruct((B, HID), f32),
        jax.ShapeDtypeStruct((1, 1), f32),
        jax.ShapeDtypeStruct((1, 1), f32),
    ]
    h, l1, l2 = pl.pallas_call(
        _fwd_kernel,
        out_shape=out_shapes,
        scratch_shapes=[pltpu.VMEM((L, B, HID), f32)],
    )(
        cov, p['cov_W1'], p['cov_b1'][None, :], p['cov_W2'], p['cov_b2'][None, :],
        Wrz, p['ode_Whh'], Wpg, bpg,
        p['p_W1'], p['p_b1'][None, :], p['p_W2'], p['p_b2'][None, :],
        Axm, Ap, p['gru_Wih'], p['gru_bih'][None, :],
        X3, M3, dtm, dtv)
    l1 = l1[0, 0]
    l2 = l2[0, 0]
    loss = l1 + MIX * l2
    nll = l1 / (L * B * D)
    return h, loss, nll, l1, l2


# speedup vs baseline: 7.0885x; 1.0510x over previous
import math

import jax
import jax.numpy as jnp
from jax.experimental import pallas as pl
from jax.experimental.pallas import tpu as pltpu

B, L, LV = 16, 256, 4
D = 64
HID = 256
PHID = 128
PREP = 16
MIX = 0.1
LOG_LIK_C = math.log(math.sqrt(2.0 * math.pi))
OBS_STD = 0.01
LOG_S2 = math.log(OBS_STD)
INV_2S2 = 1.0 / (2.0 * OBS_STD * OBS_STD)
LVP = 8


def _fwd_kernel(cov_ref, covW1_ref, covb1_ref, covW2_ref, covb2_ref,
                Wrz_ref, Whh_ref, Wgh_ref, bgh_ref,
                pW1_ref, pb1_ref, pW2_ref, pb2_ref,
                Axm_ref, Aml_ref, Ae_ref, Wih_ref, bih_ref,
                X_ref, M_ref, dtm_ref, dtv_ref,
                h_ref, l1_ref, l2_ref, Hall_ref):
    a = jnp.maximum(cov_ref[...] @ covW1_ref[...] + covb1_ref[...], 0.0)
    h0 = jnp.tanh(a @ covW2_ref[...] + covb2_ref[...])

    Wrz = Wrz_ref[...]
    Whh = Whh_ref[...]
    Wgh = Wgh_ref[...]
    bgh = bgh_ref[...]
    pW1 = pW1_ref[...]
    pb1 = pb1_ref[...]
    pW2 = pW2_ref[...]
    pb2 = pb2_ref[...]
    Axm = Axm_ref[...]
    Aml = Aml_ref[...]
    Ae = Ae_ref[...]
    Wih = Wih_ref[...]
    bih = bih_ref[...]
    two_c = 2.0 * LOG_LIK_C

    def step(t, carry):
        h, pc, acc1 = carry
        Xt = X_ref[t]
        Mt = M_ref[t]
        base = jnp.concatenate([Xt * Mt, Mt], axis=1) @ Axm
        rz = jax.nn.sigmoid(pc)
        r = rz[:, :HID]
        z = rz[:, HID:]
        u = jnp.tanh((r * h) @ Whh)
        h = h + (1.0 - z) * (u - h) * dtm_ref[t]
        a1 = jnp.maximum(h @ pW1 + pb1, 0.0)
        gh = h @ Wgh + bgh
        p = a1 @ pW2 + pb2
        mean = p[:, :D]
        logvar = p[:, D:]
        error = (Xt - mean) * jnp.exp(-0.5 * logvar)
        acc1 = acc1 + (error * error + logvar + two_c) * Mt
        pre = base + jnp.concatenate([mean * Mt, logvar * Mt], axis=1) @ Aml
        gi = jnp.maximum(pre + (error * Mt) @ Ae, 0.0)
        gg = gi @ Wih + bih
        rg = jax.nn.sigmoid(gg[:, :HID] + gh[:, :HID])
        zg = jax.nn.sigmoid(gg[:, HID:2 * HID] + gh[:, HID:2 * HID])
        n = jnp.tanh(gg[:, 2 * HID:] + rg * gh[:, 2 * HID:])
        h = (1.0 - zg) * n + zg * h
        Hall_ref[t] = h
        pc = h @ Wrz
        return (h, pc, acc1)

    zero = jnp.zeros((B, D), jnp.float32)
    pc0 = h0 @ Wrz
    h, pc, acc1 = jax.lax.fori_loop(0, L, step, (h0, pc0, zero))

    def vstep(j, carry):
        h, pc = carry
        rz = jax.nn.sigmoid(pc)
        r = rz[:, :HID]
        z = rz[:, HID:]
        u = jnp.tanh((r * h) @ Whh)
        h = h + (1.0 - z) * (u - h) * dtv_ref[j]
        return (h, h @ Wrz)

    hf, _ = jax.lax.fori_loop(0, LVP, vstep, (h, pc))
    h_ref[...] = hf

    H = Hall_ref[...].reshape(L * B, HID)
    A2 = jnp.maximum(H @ pW1_ref[...] + pb1_ref[...], 0.0)
    P2 = A2 @ pW2 + pb2
    m2 = P2[:, :D]
    v2 = P2[:, D:]
    Xf = X_ref[...].reshape(L * B, D)
    Mf = M_ref[...].reshape(L * B, D)
    kl = (LOG_S2 - 0.5) - 0.5 * v2 + (jnp.exp(v2) + jnp.square(m2 - Xf)) * INV_2S2
    l1_ref[...] = jnp.reshape(0.5 * jnp.sum(acc1), (1, 1))
    l2_ref[...] = jnp.reshape(jnp.sum(kl * Mf), (1, 1))


def kernel(times, num_obs, X, M, delta_t, cov, val_times, params):
    p = params
    f32 = jnp.float32
    dtm = jnp.concatenate([times[:, :1], times[:, 1:] - times[:, :-1]], axis=1)
    dtm = dtm.T[:, :, None]
    vt = jnp.concatenate([times[:, -1:], val_times], axis=1)
    dtv = (vt[:, 1:] - vt[:, :-1]).T[:, :, None]
    dtv = jnp.concatenate([dtv, jnp.zeros((LVP - LV, B, 1), f32)], axis=0)

    X3 = X.reshape(B, L, D).transpose(1, 0, 2)
    M3 = M.reshape(B, L, D).transpose(1, 0, 2)

    Wrz = jnp.concatenate([p['ode_Whr'], p['ode_Whz']], axis=1)

    eye = jnp.eye(D, dtype=f32)
    wp = p['w_prep']
    blk = [(eye[:, :, None] * wp[None, :, k, :]).reshape(D, D * PREP)
           for k in range(4)]
    bblk = (eye[:, :, None] * p['bias_prep'][None, :, :]).reshape(D, D * PREP)
    Axm = jnp.concatenate([blk[0], bblk], axis=0)
    Aml = jnp.concatenate([blk[1], blk[2]], axis=0)
    Ae = blk[3]

    out_shapes = [
        jax.ShapeDtypeStruct((B, HID), f32),
        jax.ShapeDtypeStruct((1, 1), f32),
        jax.ShapeDtypeStruct((1, 1), f32),
    ]
    h, l1, l2 = pl.pallas_call(
        _fwd_kernel,
        out_shape=out_shapes,
        scratch_shapes=[pltpu.VMEM((L, B, HID), f32)],
    )(
        cov, p['cov_W1'], p['cov_b1'][None, :], p['cov_W2'], p['cov_b2'][None, :],
        Wrz, p['ode_Whh'], p['gru_Whh'], p['gru_bhh'][None, :],
        p['p_W1'], p['p_b1'][None, :], p['p_W2'], p['p_b2'][None, :],
        Axm, Aml, Ae, p['gru_Wih'], p['gru_bih'][None, :],
        X3, M3, dtm, dtv)
    l1 = l1[0, 0]
    l2 = l2[0, 0]
    loss = l1 + MIX * l2
    nll = l1 / (L * B * D)
    return h, loss, nll, l1, l2
